# Initial kernel scaffold; baseline (speedup 1.0000x reference)
import jax
import jax.numpy as jnp
from jax.experimental import pallas as pl


def kernel(edge_index, edge_weight, emb_users, emb_items, W, b):
    raise NotImplementedError("write your pallas kernel here")


# speedup vs baseline: 4.5229x; 4.5229x over previous
import functools

import jax
import jax.numpy as jnp
from jax import lax
from jax.experimental import pallas as pl
from jax.experimental.pallas import tpu as pltpu
from jax.experimental.pallas import tpu_sc as plsc

N = 50000
E = 800000
D = 64
NPAD = 50048
NB = E // 128
HALF = N // 2
ACC_ROWS = 25600
WB = 120

_f32 = jnp.float32
_i32 = jnp.int32


@functools.cache
def _mesh():
    return plsc.VectorSubcoreMesh(
        core_axis_name="c", subcore_axis_name="s", num_cores=2,
        num_subcores=16)


def _zero_fill(buf, rows):
    z = jnp.zeros((16,), _f32)

    def body(r, _):
        for j in range(4):
            buf[r, pl.ds(j * 16, 16)] = z
        return 0

    lax.fori_loop(0, rows, body, 0)


@functools.cache
def _deg_kernel():
    return functools.partial(
        pl.kernel,
        out_type=jax.ShapeDtypeStruct((2 * NPAD,), _f32),
        mesh=_mesh(),
        compiler_params=pltpu.CompilerParams(
            needs_layout_passes=False, use_tc_tiling_on_sc=False),
        scratch_types=[
            pltpu.VMEM((128,), _i32),
            pltpu.VMEM((128,), _f32),
            pltpu.VMEM((3136,), _f32),
            pltpu.VMEM_SHARED((NPAD,), _f32),
        ],
    )(_deg_body)


def _deg_body(cols_hbm, ew_hbm, deg_out, col_v, ew_v, zb, deg_sp):
    c = lax.axis_index("c")
    s = lax.axis_index("s")
    z = jnp.zeros((16,), _f32)

    def zb_body(i, _):
        zb[pl.ds(i * 16, 16)] = z
        return 0

    lax.fori_loop(0, 196, zb_body, 0)
    pltpu.sync_copy(zb.at[pl.ds(0, 3128)], deg_sp.at[pl.ds(s * 3128, 3128)])
    plsc.subcore_barrier()

    w = c * 16 + s
    start = w * 195 + jnp.minimum(w, 10)
    cnt = 195 + jnp.where(w < 10, 1, 0)

    def body(t, _):
        eoff = (start + t) * 128
        pltpu.sync_copy(cols_hbm.at[pl.ds(eoff, 128)], col_v)
        pltpu.sync_copy(ew_hbm.at[pl.ds(eoff, 128)], ew_v)
        pltpu.sync_copy(ew_v, deg_sp.at[col_v], add=True)
        return 0

    lax.fori_loop(0, cnt, body, 0)
    plsc.subcore_barrier()
    pltpu.sync_copy(deg_sp.at[pl.ds(s * 3128, 3128)], zb.at[pl.ds(0, 3128)])
    pltpu.sync_copy(zb.at[pl.ds(0, 3128)],
                    deg_out.at[pl.ds(c * NPAD + s * 3128, 3128)])


def _dis_body(deg_ref, dis_ref):
    d = deg_ref[0] + deg_ref[1]
    dis_ref[...] = jnp.where(d > 0, lax.rsqrt(d), 0.0)


def _dis(deg_p):
    out = pl.pallas_call(
        _dis_body,
        out_shape=jax.ShapeDtypeStruct((391, 128), _f32),
    )(deg_p.reshape(2, 391, 128))
    return out.reshape(NPAD)


def _layer_body(first, *refs):
    if first:
        (rows_hbm, cols_hbm, ew_hbm, dis_hbm, x_hbm, y_out, norm_out,
         rows_v, cols_v, scat_v, ew_v, norm_v, nm_v, drbuf, dcbuf,
         xbuf, wb, dis_sp, acc, gsem) = refs
    else:
        (rows_hbm, cols_hbm, norm_hbm, x_hbm, y_out,
         rows_v, cols_v, scat_v, norm_v, nm_v, xbuf, wb, acc, gsem) = refs

    c = lax.axis_index("c")
    s = lax.axis_index("s")
    base = c * HALF

    if first:
        @pl.when(s == 0)
        def _():
            pltpu.sync_copy(dis_hbm, dis_sp)

    _zero_fill(wb, WB)
    arow = s * 1600

    def zbody(k, _):
        pltpu.sync_copy(wb, acc.at[pl.ds(arow + k * WB, WB)])
        return 0

    lax.fori_loop(0, 13, zbody, 0)
    pltpu.sync_copy(wb.at[pl.ds(0, 40)], acc.at[pl.ds(arow + 1560, 40)])
    plsc.subcore_barrier()

    start = s * 390 + jnp.minimum(s, 10)
    cnt = 390 + jnp.where(s < 10, 1, 0)

    def body(t, _):
        eoff = (start + t) * 128
        pltpu.sync_copy(rows_hbm.at[pl.ds(eoff, 128)], rows_v)
        pltpu.sync_copy(cols_hbm.at[pl.ds(eoff, 128)], cols_v)
        if first:
            pltpu.sync_copy(ew_hbm.at[pl.ds(eoff, 128)], ew_v)
        else:
            pltpu.sync_copy(norm_hbm.at[pl.ds(eoff, 128)], norm_v)
        pltpu.async_copy(x_hbm.at[rows_v], xbuf, gsem).wait()
        if first:
            pltpu.async_copy(dis_sp.at[rows_v], drbuf, gsem).wait()
            pltpu.async_copy(dis_sp.at[cols_v], dcbuf, gsem).wait()

        for i in range(8):
            sl = pl.ds(i * 16, 16)
            c16 = cols_v[sl]
            lc = c16 - base
            valid = (lc >= 0) & (lc < HALF)
            if first:
                nv = drbuf[sl] * ew_v[sl] * dcbuf[sl]
                norm_v[sl] = nv
            else:
                nv = norm_v[sl]
            scat_v[sl] = jnp.where(valid, lc, 0)
            nm_v[sl] = jnp.where(valid, nv, 0.0)

        if first:
            @pl.when(c == 0)
            def _():
                pltpu.sync_copy(norm_v, norm_out.at[pl.ds(eoff, 128)])

        def sbody(g, _):
            nv16 = nm_v[pl.ds(g * 16, 16)]
            for l in range(16):
                sc = nv16[l]
                e = g * 16 + l
                for j in range(4):
                    sl2 = pl.ds(j * 16, 16)
                    xbuf[e, sl2] = xbuf[e, sl2] * sc
            return 0

        lax.fori_loop(0, 8, sbody, 0)
        pltpu.sync_copy(xbuf, acc.at[scat_v], add=True)
        return 0

    lax.fori_loop(0, cnt, body, 0)
    plsc.subcore_barrier()

    def wbody(k, _):
        rbase = s * 1560 + k * WB
        pltpu.sync_copy(acc.at[pl.ds(rbase, WB)], wb)

        def rbody(r, _):
            for j in range(4):
                sl2 = pl.ds(j * 16, 16)
                wb[r, sl2] = jnp.maximum(wb[r, sl2], 0.0)
            return 0

        lax.fori_loop(0, WB, rbody, 0)
        pltpu.sync_copy(wb, y_out.at[pl.ds(base + rbase, WB)])
        return 0

    lax.fori_loop(0, 13, wbody, 0)

    @pl.when(s == 15)
    def _():
        pltpu.sync_copy(acc.at[pl.ds(24960, 40)], wb.at[pl.ds(0, 40)])

        def rbody(r, _):
            for j in range(4):
                sl2 = pl.ds(j * 16, 16)
                wb[r, sl2] = jnp.maximum(wb[r, sl2], 0.0)
            return 0

        lax.fori_loop(0, 40, rbody, 0)
        pltpu.sync_copy(wb.at[pl.ds(0, 40)],
                        y_out.at[pl.ds(base + 24960, 40)])


@functools.cache
def _layer1():
    return functools.partial(
        pl.kernel,
        out_type=(jax.ShapeDtypeStruct((N, D), _f32),
                  jax.ShapeDtypeStruct((E,), _f32)),
        mesh=_mesh(),
        compiler_params=pltpu.CompilerParams(
            needs_layout_passes=False, use_tc_tiling_on_sc=False),
        scratch_types=[
            pltpu.VMEM((128,), _i32),
            pltpu.VMEM((128,), _i32),
            pltpu.VMEM((128,), _i32),
            pltpu.VMEM((128,), _f32),
            pltpu.VMEM((128,), _f32),
            pltpu.VMEM((128,), _f32),
            pltpu.VMEM((128,), _f32),
            pltpu.VMEM((128,), _f32),
            pltpu.VMEM((128, D), _f32),
            pltpu.VMEM((WB, D), _f32),
            pltpu.VMEM_SHARED((NPAD,), _f32),
            pltpu.VMEM_SHARED((ACC_ROWS, D), _f32),
            pltpu.SemaphoreType.DMA,
        ],
    )(functools.partial(_layer_body, True))


@functools.cache
def _layer2():
    return functools.partial(
        pl.kernel,
        out_type=jax.ShapeDtypeStruct((N, D), _f32),
        mesh=_mesh(),
        compiler_params=pltpu.CompilerParams(
            needs_layout_passes=False, use_tc_tiling_on_sc=False),
        scratch_types=[
            pltpu.VMEM((128,), _i32),
            pltpu.VMEM((128,), _i32),
            pltpu.VMEM((128,), _i32),
            pltpu.VMEM((128,), _f32),
            pltpu.VMEM((128,), _f32),
            pltpu.VMEM((128, D), _f32),
            pltpu.VMEM((WB, D), _f32),
            pltpu.VMEM_SHARED((ACC_ROWS, D), _f32),
            pltpu.SemaphoreType.DMA,
        ],
    )(functools.partial(_layer_body, False))


def _final_body(x0_ref, y1_ref, y2_ref, w_ref, b_ref, out_ref):
    xs = (x0_ref[...] + y1_ref[...] + y2_ref[...]) * (1.0 / 3.0)
    out_ref[...] = lax.dot_general(
        xs, w_ref[...], (((1,), (1,)), ((), ())),
        preferred_element_type=_f32) + b_ref[...]


def _final(x0, y1, y2, W, b):
    grid = 125
    blk = N // grid
    return pl.pallas_call(
        _final_body,
        grid=(grid,),
        in_specs=[
            pl.BlockSpec((blk, D), lambda i: (i, 0)),
            pl.BlockSpec((blk, D), lambda i: (i, 0)),
            pl.BlockSpec((blk, D), lambda i: (i, 0)),
            pl.BlockSpec((D, D), lambda i: (0, 0)),
            pl.BlockSpec((1, D), lambda i: (0, 0)),
        ],
        out_specs=pl.BlockSpec((blk, D), lambda i: (i, 0)),
        out_shape=jax.ShapeDtypeStruct((N, D), _f32),
    )(x0, y1, y2, W, b.reshape(1, D))


def kernel(edge_index, edge_weight, emb_users, emb_items, W, b):
    rows1d = edge_index[0].astype(_i32)
    cols1d = edge_index[1].astype(_i32)
    x0 = jnp.concatenate([emb_users, emb_items], axis=0)

    deg_p = _deg_kernel()(cols1d, edge_weight)
    dis = _dis(deg_p)
    y1, norm1d = _layer1()(rows1d, cols1d, edge_weight, dis, x0)
    y2 = _layer2()(rows1d, cols1d, norm1d, y1)
    out = _final(x0, y1, y2, W, b)
    nu = emb_users.shape[0]
    return (out[:nu], emb_users, out[nu:], emb_items)


# speedup vs baseline: 5.0854x; 1.1244x over previous
import functools

import jax
import jax.numpy as jnp
from jax import lax
from jax.experimental import pallas as pl
from jax.experimental.pallas import tpu as pltpu
from jax.experimental.pallas import tpu_sc as plsc

N = 50000
E = 800000
D = 64
NPAD = 50048
NB = E // 128
HALF = N // 2
ACC_ROWS = 25600
WB = 80

_f32 = jnp.float32
_i32 = jnp.int32


@functools.cache
def _mesh():
    return plsc.VectorSubcoreMesh(
        core_axis_name="c", subcore_axis_name="s", num_cores=2,
        num_subcores=16)


def _zero_fill(buf, rows):
    z = jnp.zeros((16,), _f32)

    def body(r, _):
        for j in range(4):
            buf[r, pl.ds(j * 16, 16)] = z
        return 0

    lax.fori_loop(0, rows, body, 0)


@functools.cache
def _deg_kernel():
    return functools.partial(
        pl.kernel,
        out_type=jax.ShapeDtypeStruct((2 * NPAD,), _f32),
        mesh=_mesh(),
        compiler_params=pltpu.CompilerParams(
            needs_layout_passes=False, use_tc_tiling_on_sc=False),
        scratch_types=[
            pltpu.VMEM((128,), _i32),
            pltpu.VMEM((128,), _f32),
            pltpu.VMEM((3136,), _f32),
            pltpu.VMEM_SHARED((NPAD,), _f32),
        ],
    )(_deg_body)


def _deg_body(cols_hbm, ew_hbm, deg_out, col_v, ew_v, zb, deg_sp):
    c = lax.axis_index("c")
    s = lax.axis_index("s")
    z = jnp.zeros((16,), _f32)

    def zb_body(i, _):
        zb[pl.ds(i * 16, 16)] = z
        return 0

    lax.fori_loop(0, 196, zb_body, 0)
    pltpu.sync_copy(zb.at[pl.ds(0, 3128)], deg_sp.at[pl.ds(s * 3128, 3128)])
    plsc.subcore_barrier()

    w = c * 16 + s
    start = w * 195 + jnp.minimum(w, 10)
    cnt = 195 + jnp.where(w < 10, 1, 0)

    def body(t, _):
        eoff = (start + t) * 128
        pltpu.sync_copy(cols_hbm.at[pl.ds(eoff, 128)], col_v)
        pltpu.sync_copy(ew_hbm.at[pl.ds(eoff, 128)], ew_v)
        pltpu.sync_copy(ew_v, deg_sp.at[col_v], add=True)
        return 0

    lax.fori_loop(0, cnt, body, 0)
    plsc.subcore_barrier()
    pltpu.sync_copy(deg_sp.at[pl.ds(s * 3128, 3128)], zb.at[pl.ds(0, 3128)])
    pltpu.sync_copy(zb.at[pl.ds(0, 3128)],
                    deg_out.at[pl.ds(c * NPAD + s * 3128, 3128)])


def _dis_body(deg_ref, dis_ref):
    d = deg_ref[0] + deg_ref[1]
    dis_ref[...] = jnp.where(d > 0, lax.rsqrt(d), 0.0)


def _dis(deg_p):
    out = pl.pallas_call(
        _dis_body,
        out_shape=jax.ShapeDtypeStruct((391, 128), _f32),
    )(deg_p.reshape(2, 391, 128))
    return out.reshape(NPAD)


SUP = 256
NSUP = E // SUP


def _layer_body(first, *refs):
    if first:
        (rows_hbm, cols_hbm, ew_hbm, dis_hbm, x_hbm, y_out, norm_out,
         rows_v, cols_v, scat_v, ew_v, norm_v, nm_v, drbuf, dcbuf,
         xbuf, wb, dis_sp, acc, esem, gsem, ssem) = refs
    else:
        (rows_hbm, cols_hbm, norm_hbm, x_hbm, y_out,
         rows_v, cols_v, scat_v, norm_v, nm_v, xbuf, wb, acc,
         esem, gsem, ssem) = refs

    c = lax.axis_index("c")
    s = lax.axis_index("s")
    base = c * HALF

    if first:
        @pl.when(s == 0)
        def _():
            pltpu.sync_copy(dis_hbm, dis_sp)

    _zero_fill(wb, WB)
    arow = s * 1600

    def zbody(k, _):
        pltpu.sync_copy(wb, acc.at[pl.ds(arow + k * WB, WB)])
        return 0

    lax.fori_loop(0, 20, zbody, 0)
    plsc.subcore_barrier()

    start = s * 195 + jnp.minimum(s, 5)
    cnt = 195 + jnp.where(s < 5, 1, 0)

    def body(t, _):
        par = t % 2
        u = start + t
        pltpu.sync_copy(rows_hbm.at[pl.ds(2 * u, 2)], rows_v.at[par])
        pltpu.sync_copy(cols_hbm.at[pl.ds(2 * u, 2)], cols_v.at[par])
        if first:
            pltpu.sync_copy(ew_hbm.at[pl.ds(u * SUP, SUP)], ew_v.at[par])
        else:
            pltpu.sync_copy(norm_hbm.at[pl.ds(u * SUP, SUP)], norm_v.at[par])

        for k in range(2):
            pltpu.async_copy(x_hbm.at[rows_v.at[par, k]],
                             xbuf.at[pl.ds(k * 128, 128)], gsem).wait()
        if first:
            for k in range(2):
                pltpu.async_copy(dis_sp.at[rows_v.at[par, k]],
                                 drbuf.at[k], gsem).wait()
                pltpu.async_copy(dis_sp.at[cols_v.at[par, k]],
                                 dcbuf.at[k], gsem).wait()

        for k in range(2):
            for i in range(8):
                sl = pl.ds(i * 16, 16)
                e0 = k * 128 + i * 16
                c16 = cols_v[par, k, sl]
                lc = c16 - base
                valid = (lc >= 0) & (lc < HALF)
                if first:
                    nv = (drbuf[k, sl] * ew_v[par, pl.ds(e0, 16)]
                          * dcbuf[k, sl])
                    norm_v[par, pl.ds(e0, 16)] = nv
                else:
                    nv = norm_v[par, pl.ds(e0, 16)]
                scat_v[k, sl] = jnp.where(valid, lc, 0)
                nm_v[pl.ds(e0, 16)] = jnp.where(valid, nv, 0.0)

        if first:
            @pl.when(c == 0)
            def _():
                pltpu.sync_copy(norm_v.at[par],
                                norm_out.at[pl.ds(u * SUP, SUP)])

        def sbody(g, _):
            nv16 = nm_v[pl.ds(g * 16, 16)]
            for l in range(16):
                sc = nv16[l]
                e = g * 16 + l
                for j in range(4):
                    sl2 = pl.ds(j * 16, 16)
                    xbuf[e, sl2] = xbuf[e, sl2] * sc
            return 0

        lax.fori_loop(0, 16, sbody, 0)
        for k in range(2):
            pltpu.sync_copy(xbuf.at[pl.ds(k * 128, 128)],
                            acc.at[scat_v.at[k]], add=True)
        return 0

    lax.fori_loop(0, cnt, body, 0)
    plsc.subcore_barrier()

    def _relu_rows(n):
        def rbody(r, _):
            for j in range(4):
                sl2 = pl.ds(j * 16, 16)
                wb[r, sl2] = jnp.maximum(wb[r, sl2], 0.0)
            return 0

        lax.fori_loop(0, n, rbody, 0)

    def wbody(k, _):
        rbase = s * 1560 + k * WB
        pltpu.sync_copy(acc.at[pl.ds(rbase, WB)], wb)
        _relu_rows(WB)
        pltpu.sync_copy(wb, y_out.at[pl.ds(base + rbase, WB)])
        return 0

    lax.fori_loop(0, 19, wbody, 0)
    rbase = s * 1560 + 1520
    pltpu.sync_copy(acc.at[pl.ds(rbase, 40)], wb.at[pl.ds(0, 40)])
    _relu_rows(40)
    pltpu.sync_copy(wb.at[pl.ds(0, 40)], y_out.at[pl.ds(base + rbase, 40)])

    @pl.when(s == 15)
    def _():
        pltpu.sync_copy(acc.at[pl.ds(24960, 40)], wb.at[pl.ds(0, 40)])
        _relu_rows(40)
        pltpu.sync_copy(wb.at[pl.ds(0, 40)],
                        y_out.at[pl.ds(base + 24960, 40)])


@functools.cache
def _layer1():
    return functools.partial(
        pl.kernel,
        out_type=(jax.ShapeDtypeStruct((N, D), _f32),
                  jax.ShapeDtypeStruct((E,), _f32)),
        mesh=_mesh(),
        compiler_params=pltpu.CompilerParams(
            needs_layout_passes=False, use_tc_tiling_on_sc=False),
        scratch_types=[
            pltpu.VMEM((2, 2, 128), _i32),
            pltpu.VMEM((2, 2, 128), _i32),
            pltpu.VMEM((2, 128), _i32),
            pltpu.VMEM((2, SUP), _f32),
            pltpu.VMEM((2, SUP), _f32),
            pltpu.VMEM((SUP,), _f32),
            pltpu.VMEM((2, 128), _f32),
            pltpu.VMEM((2, 128), _f32),
            pltpu.VMEM((SUP, D), _f32),
            pltpu.VMEM((WB, D), _f32),
            pltpu.VMEM_SHARED((NPAD,), _f32),
            pltpu.VMEM_SHARED((ACC_ROWS, D), _f32),
            pltpu.SemaphoreType.DMA,
            pltpu.SemaphoreType.DMA,
            pltpu.SemaphoreType.DMA,
        ],
    )(functools.partial(_layer_body, True))


@functools.cache
def _layer2():
    return functools.partial(
        pl.kernel,
        out_type=jax.ShapeDtypeStruct((N, D), _f32),
        mesh=_mesh(),
        compiler_params=pltpu.CompilerParams(
            needs_layout_passes=False, use_tc_tiling_on_sc=False),
        scratch_types=[
            pltpu.VMEM((2, 2, 128), _i32),
            pltpu.VMEM((2, 2, 128), _i32),
            pltpu.VMEM((2, 128), _i32),
            pltpu.VMEM((2, SUP), _f32),
            pltpu.VMEM((SUP,), _f32),
            pltpu.VMEM((SUP, D), _f32),
            pltpu.VMEM((WB, D), _f32),
            pltpu.VMEM_SHARED((ACC_ROWS, D), _f32),
            pltpu.SemaphoreType.DMA,
            pltpu.SemaphoreType.DMA,
            pltpu.SemaphoreType.DMA,
        ],
    )(functools.partial(_layer_body, False))


def _final_body(x0_ref, y1_ref, y2_ref, w_ref, b_ref, out_ref):
    xs = (x0_ref[...] + y1_ref[...] + y2_ref[...]) * (1.0 / 3.0)
    out_ref[...] = lax.dot_general(
        xs, w_ref[...], (((1,), (1,)), ((), ())),
        preferred_element_type=_f32) + b_ref[...]


def _final(x0, y1, y2, W, b):
    grid = 125
    blk = N // grid
    return pl.pallas_call(
        _final_body,
        grid=(grid,),
        in_specs=[
            pl.BlockSpec((blk, D), lambda i: (i, 0)),
            pl.BlockSpec((blk, D), lambda i: (i, 0)),
            pl.BlockSpec((blk, D), lambda i: (i, 0)),
            pl.BlockSpec((D, D), lambda i: (0, 0)),
            pl.BlockSpec((1, D), lambda i: (0, 0)),
        ],
        out_specs=pl.BlockSpec((blk, D), lambda i: (i, 0)),
        out_shape=jax.ShapeDtypeStruct((N, D), _f32),
    )(x0, y1, y2, W, b.reshape(1, D))


def kernel(edge_index, edge_weight, emb_users, emb_items, W, b):
    rows1d = edge_index[0].astype(_i32)
    cols1d = edge_index[1].astype(_i32)
    rows2d = rows1d.reshape(NB, 128)
    cols2d = cols1d.reshape(NB, 128)
    x0 = jnp.concatenate([emb_users, emb_items], axis=0)

    deg_p = _deg_kernel()(cols1d, edge_weight)
    dis = _dis(deg_p)
    y1, norm1d = _layer1()(rows2d, cols2d, edge_weight, dis, x0)
    y2 = _layer2()(rows2d, cols2d, norm1d, y1)
    out = _final(x0, y1, y2, W, b)
    nu = emb_users.shape[0]
    return (out[:nu], emb_users, out[nu:], emb_items)


# speedup vs baseline: 6.2691x; 1.2328x over previous
import functools

import jax
import jax.numpy as jnp
from jax import lax
from jax.experimental import pallas as pl
from jax.experimental.pallas import tpu as pltpu
from jax.experimental.pallas import tpu_sc as plsc

N = 50000
E = 800000
D = 64
NPAD = 50048
NB = E // 128
HALF = N // 2
ACC_ROWS = 25600
WB = 80

_f32 = jnp.float32
_i32 = jnp.int32


@functools.cache
def _mesh():
    return plsc.VectorSubcoreMesh(
        core_axis_name="c", subcore_axis_name="s", num_cores=2,
        num_subcores=16)


def _zero_fill(buf, rows):
    z = jnp.zeros((16,), _f32)

    def body(r, _):
        for j in range(4):
            buf[r, pl.ds(j * 16, 16)] = z
        return 0

    lax.fori_loop(0, rows, body, 0)


@functools.cache
def _deg_kernel():
    return functools.partial(
        pl.kernel,
        out_type=jax.ShapeDtypeStruct((2 * NPAD,), _f32),
        mesh=_mesh(),
        compiler_params=pltpu.CompilerParams(
            needs_layout_passes=False, use_tc_tiling_on_sc=False),
        scratch_types=[
            pltpu.VMEM((128,), _i32),
            pltpu.VMEM((128,), _f32),
            pltpu.VMEM((3136,), _f32),
            pltpu.VMEM_SHARED((NPAD,), _f32),
        ],
    )(_deg_body)


def _deg_body(cols_hbm, ew_hbm, deg_out, col_v, ew_v, zb, deg_sp):
    c = lax.axis_index("c")
    s = lax.axis_index("s")
    z = jnp.zeros((16,), _f32)

    def zb_body(i, _):
        zb[pl.ds(i * 16, 16)] = z
        return 0

    lax.fori_loop(0, 196, zb_body, 0)
    pltpu.sync_copy(zb.at[pl.ds(0, 3128)], deg_sp.at[pl.ds(s * 3128, 3128)])
    plsc.subcore_barrier()

    w = c * 16 + s
    start = w * 195 + jnp.minimum(w, 10)
    cnt = 195 + jnp.where(w < 10, 1, 0)

    def body(t, _):
        eoff = (start + t) * 128
        pltpu.sync_copy(cols_hbm.at[pl.ds(eoff, 128)], col_v)
        pltpu.sync_copy(ew_hbm.at[pl.ds(eoff, 128)], ew_v)
        pltpu.sync_copy(ew_v, deg_sp.at[col_v], add=True)
        return 0

    lax.fori_loop(0, cnt, body, 0)
    plsc.subcore_barrier()
    pltpu.sync_copy(deg_sp.at[pl.ds(s * 3128, 3128)], zb.at[pl.ds(0, 3128)])
    pltpu.sync_copy(zb.at[pl.ds(0, 3128)],
                    deg_out.at[pl.ds(c * NPAD + s * 3128, 3128)])


def _dis_body(deg_ref, dis_ref):
    d = deg_ref[0] + deg_ref[1]
    dis_ref[...] = jnp.where(d > 0, lax.rsqrt(d), 0.0)


def _dis(deg_p):
    out = pl.pallas_call(
        _dis_body,
        out_shape=jax.ShapeDtypeStruct((391, 128), _f32),
    )(deg_p.reshape(2, 391, 128))
    return out.reshape(NPAD)


SUP = 256
NSUP = E // SUP


def _layer_body(first, *refs):
    if first:
        (rows_hbm, cols_hbm, ew_hbm, dis_hbm, x_hbm, y_out, norm_out,
         rows_v, cols_v, scat_v, ew_v, norm_v, nm_v, drbuf, dcbuf,
         xbuf, wb, dis_sp, acc, esem, gsem, ssem) = refs
    else:
        (rows_hbm, cols_hbm, norm_hbm, x_hbm, y_out,
         rows_v, cols_v, scat_v, norm_v, nm_v, xbuf, wb, acc,
         esem, gsem, ssem) = refs

    c = lax.axis_index("c")
    s = lax.axis_index("s")
    base = c * HALF

    if first:
        @pl.when(s == 0)
        def _():
            pltpu.sync_copy(dis_hbm, dis_sp)

    _zero_fill(wb, WB)
    arow = s * 1600

    def zbody(k, _):
        pltpu.sync_copy(wb, acc.at[pl.ds(arow + k * WB, WB)])
        return 0

    lax.fori_loop(0, 20, zbody, 0)
    plsc.subcore_barrier()

    start = s * 195 + jnp.minimum(s, 5)
    cnt = 195 + jnp.where(s < 5, 1, 0)

    def body(t, _):
        par = t % 2
        u = start + t
        ew = [pltpu.async_copy(rows_hbm.at[pl.ds(2 * u, 2)],
                               rows_v.at[par], esem.at[0]),
              pltpu.async_copy(cols_hbm.at[pl.ds(2 * u, 2)],
                               cols_v.at[par], esem.at[1])]
        if first:
            ew.append(pltpu.async_copy(ew_hbm.at[pl.ds(u * SUP, SUP)],
                                       ew_v.at[par], esem.at[2]))
        else:
            ew.append(pltpu.async_copy(norm_hbm.at[pl.ds(u * SUP, SUP)],
                                       norm_v.at[par], esem.at[2]))
        for wd in ew:
            wd.wait()

        waits = [pltpu.async_copy(x_hbm.at[rows_v.at[par, k]],
                                  xbuf.at[pl.ds(k * 128, 128)], gsem.at[k])
                 for k in range(2)]
        if first:
            for k in range(2):
                waits.append(pltpu.async_copy(
                    dis_sp.at[rows_v.at[par, k]], drbuf.at[k],
                    gsem.at[2 + k]))
                waits.append(pltpu.async_copy(
                    dis_sp.at[cols_v.at[par, k]], dcbuf.at[k],
                    gsem.at[4 + k]))
        for wd in waits:
            wd.wait()

        for k in range(2):
            for i in range(8):
                sl = pl.ds(i * 16, 16)
                e0 = k * 128 + i * 16
                c16 = cols_v[par, k, sl]
                lc = c16 - base
                valid = (lc >= 0) & (lc < HALF)
                if first:
                    nv = (drbuf[k, sl] * ew_v[par, pl.ds(e0, 16)]
                          * dcbuf[k, sl])
                    norm_v[par, pl.ds(e0, 16)] = nv
                else:
                    nv = norm_v[par, pl.ds(e0, 16)]
                scat_v[k, sl] = jnp.where(valid, lc, 0)
                nm_v[pl.ds(e0, 16)] = jnp.where(valid, nv, 0.0)

        if first:
            @pl.when(c == 0)
            def _():
                pltpu.sync_copy(norm_v.at[par],
                                norm_out.at[pl.ds(u * SUP, SUP)])

        def sbody(g, _):
            nv16 = nm_v[pl.ds(g * 16, 16)]
            for l in range(16):
                sc = nv16[l]
                e = g * 16 + l
                for j in range(4):
                    sl2 = pl.ds(j * 16, 16)
                    xbuf[e, sl2] = xbuf[e, sl2] * sc
            return 0

        lax.fori_loop(0, 16, sbody, 0)
        sw = [pltpu.async_copy(xbuf.at[pl.ds(k * 128, 128)],
                               acc.at[scat_v.at[k]], ssem.at[k], add=True)
              for k in range(2)]
        for wd in sw:
            wd.wait()
        return 0

    lax.fori_loop(0, cnt, body, 0)
    plsc.subcore_barrier()

    def _relu_rows(n):
        def rbody(r, _):
            for j in range(4):
                sl2 = pl.ds(j * 16, 16)
                wb[r, sl2] = jnp.maximum(wb[r, sl2], 0.0)
            return 0

        lax.fori_loop(0, n, rbody, 0)

    def wbody(k, _):
        rbase = s * 1560 + k * WB
        pltpu.sync_copy(acc.at[pl.ds(rbase, WB)], wb)
        _relu_rows(WB)
        pltpu.sync_copy(wb, y_out.at[pl.ds(base + rbase, WB)])
        return 0

    lax.fori_loop(0, 19, wbody, 0)
    rbase = s * 1560 + 1520
    pltpu.sync_copy(acc.at[pl.ds(rbase, 40)], wb.at[pl.ds(0, 40)])
    _relu_rows(40)
    pltpu.sync_copy(wb.at[pl.ds(0, 40)], y_out.at[pl.ds(base + rbase, 40)])

    @pl.when(s == 15)
    def _():
        pltpu.sync_copy(acc.at[pl.ds(24960, 40)], wb.at[pl.ds(0, 40)])
        _relu_rows(40)
        pltpu.sync_copy(wb.at[pl.ds(0, 40)],
                        y_out.at[pl.ds(base + 24960, 40)])


@functools.cache
def _layer1():
    return functools.partial(
        pl.kernel,
        out_type=(jax.ShapeDtypeStruct((N, D), _f32),
                  jax.ShapeDtypeStruct((E,), _f32)),
        mesh=_mesh(),
        compiler_params=pltpu.CompilerParams(
            needs_layout_passes=False, use_tc_tiling_on_sc=False),
        scratch_types=[
            pltpu.VMEM((2, 2, 128), _i32),
            pltpu.VMEM((2, 2, 128), _i32),
            pltpu.VMEM((2, 128), _i32),
            pltpu.VMEM((2, SUP), _f32),
            pltpu.VMEM((2, SUP), _f32),
            pltpu.VMEM((SUP,), _f32),
            pltpu.VMEM((2, 128), _f32),
            pltpu.VMEM((2, 128), _f32),
            pltpu.VMEM((SUP, D), _f32),
            pltpu.VMEM((WB, D), _f32),
            pltpu.VMEM_SHARED((NPAD,), _f32),
            pltpu.VMEM_SHARED((ACC_ROWS, D), _f32),
            pltpu.SemaphoreType.DMA((3,)),
            pltpu.SemaphoreType.DMA((6,)),
            pltpu.SemaphoreType.DMA((2,)),
        ],
    )(functools.partial(_layer_body, True))


@functools.cache
def _layer2():
    return functools.partial(
        pl.kernel,
        out_type=jax.ShapeDtypeStruct((N, D), _f32),
        mesh=_mesh(),
        compiler_params=pltpu.CompilerParams(
            needs_layout_passes=False, use_tc_tiling_on_sc=False),
        scratch_types=[
            pltpu.VMEM((2, 2, 128), _i32),
            pltpu.VMEM((2, 2, 128), _i32),
            pltpu.VMEM((2, 128), _i32),
            pltpu.VMEM((2, SUP), _f32),
            pltpu.VMEM((SUP,), _f32),
            pltpu.VMEM((SUP, D), _f32),
            pltpu.VMEM((WB, D), _f32),
            pltpu.VMEM_SHARED((ACC_ROWS, D), _f32),
            pltpu.SemaphoreType.DMA((3,)),
            pltpu.SemaphoreType.DMA((6,)),
            pltpu.SemaphoreType.DMA((2,)),
        ],
    )(functools.partial(_layer_body, False))


def _final_body(x0_ref, y1_ref, y2_ref, w_ref, b_ref, out_ref):
    xs = (x0_ref[...] + y1_ref[...] + y2_ref[...]) * (1.0 / 3.0)
    out_ref[...] = lax.dot_general(
        xs, w_ref[...], (((1,), (1,)), ((), ())),
        preferred_element_type=_f32) + b_ref[...]


def _final(x0, y1, y2, W, b):
    grid = 125
    blk = N // grid
    return pl.pallas_call(
        _final_body,
        grid=(grid,),
        in_specs=[
            pl.BlockSpec((blk, D), lambda i: (i, 0)),
            pl.BlockSpec((blk, D), lambda i: (i, 0)),
            pl.BlockSpec((blk, D), lambda i: (i, 0)),
            pl.BlockSpec((D, D), lambda i: (0, 0)),
            pl.BlockSpec((1, D), lambda i: (0, 0)),
        ],
        out_specs=pl.BlockSpec((blk, D), lambda i: (i, 0)),
        out_shape=jax.ShapeDtypeStruct((N, D), _f32),
    )(x0, y1, y2, W, b.reshape(1, D))


def kernel(edge_index, edge_weight, emb_users, emb_items, W, b):
    rows1d = edge_index[0].astype(_i32)
    cols1d = edge_index[1].astype(_i32)
    rows2d = rows1d.reshape(NB, 128)
    cols2d = cols1d.reshape(NB, 128)
    x0 = jnp.concatenate([emb_users, emb_items], axis=0)

    deg_p = _deg_kernel()(cols1d, edge_weight)
    dis = _dis(deg_p)
    y1, norm1d = _layer1()(rows2d, cols2d, edge_weight, dis, x0)
    y2 = _layer2()(rows2d, cols2d, norm1d, y1)
    out = _final(x0, y1, y2, W, b)
    nu = emb_users.shape[0]
    return (out[:nu], emb_users, out[nu:], emb_items)


# speedup vs baseline: 6.6028x; 1.0532x over previous
import functools

import jax
import jax.numpy as jnp
from jax import lax
from jax.experimental import pallas as pl
from jax.experimental.pallas import tpu as pltpu
from jax.experimental.pallas import tpu_sc as plsc

N = 50000
E = 800000
D = 64
NPAD = 50048
NB = E // 128
HALF = N // 2
ACC_ROWS = 25600
WB = 80

_f32 = jnp.float32
_i32 = jnp.int32


@functools.cache
def _mesh():
    return plsc.VectorSubcoreMesh(
        core_axis_name="c", subcore_axis_name="s", num_cores=2,
        num_subcores=16)


def _zero_fill(buf, rows):
    z = jnp.zeros((16,), _f32)

    def body(r, _):
        for j in range(4):
            buf[r, pl.ds(j * 16, 16)] = z
        return 0

    lax.fori_loop(0, rows, body, 0)


@functools.cache
def _deg_kernel():
    return functools.partial(
        pl.kernel,
        out_type=jax.ShapeDtypeStruct((2 * NPAD,), _f32),
        mesh=_mesh(),
        compiler_params=pltpu.CompilerParams(
            needs_layout_passes=False, use_tc_tiling_on_sc=False),
        scratch_types=[
            pltpu.VMEM((128,), _i32),
            pltpu.VMEM((128,), _f32),
            pltpu.VMEM((3136,), _f32),
            pltpu.VMEM_SHARED((NPAD,), _f32),
        ],
    )(_deg_body)


def _deg_body(cols_hbm, ew_hbm, deg_out, col_v, ew_v, zb, deg_sp):
    c = lax.axis_index("c")
    s = lax.axis_index("s")
    z = jnp.zeros((16,), _f32)

    def zb_body(i, _):
        zb[pl.ds(i * 16, 16)] = z
        return 0

    lax.fori_loop(0, 196, zb_body, 0)
    pltpu.sync_copy(zb.at[pl.ds(0, 3128)], deg_sp.at[pl.ds(s * 3128, 3128)])
    plsc.subcore_barrier()

    w = c * 16 + s
    start = w * 195 + jnp.minimum(w, 10)
    cnt = 195 + jnp.where(w < 10, 1, 0)

    def body(t, _):
        eoff = (start + t) * 128
        pltpu.sync_copy(cols_hbm.at[pl.ds(eoff, 128)], col_v)
        pltpu.sync_copy(ew_hbm.at[pl.ds(eoff, 128)], ew_v)
        pltpu.sync_copy(ew_v, deg_sp.at[col_v], add=True)
        return 0

    lax.fori_loop(0, cnt, body, 0)
    plsc.subcore_barrier()
    pltpu.sync_copy(deg_sp.at[pl.ds(s * 3128, 3128)], zb.at[pl.ds(0, 3128)])
    pltpu.sync_copy(zb.at[pl.ds(0, 3128)],
                    deg_out.at[pl.ds(c * NPAD + s * 3128, 3128)])


def _dis_body(deg_ref, dis_ref):
    d = deg_ref[0] + deg_ref[1]
    dis_ref[...] = jnp.where(d > 0, lax.rsqrt(d), 0.0)


def _dis(deg_p):
    out = pl.pallas_call(
        _dis_body,
        out_shape=jax.ShapeDtypeStruct((391, 128), _f32),
    )(deg_p.reshape(2, 391, 128))
    return out.reshape(NPAD)


SUP = 256
NSUP = E // SUP


def _layer_body(first, *refs):
    if first:
        (rows_hbm, cols_hbm, ew_hbm, dis_hbm, x_hbm, y_out, norm_out,
         rows_v, cols_v, scat_v, ew_v, norm_v, nm_v, drbuf, dcbuf,
         xbuf, wb, dis_sp, acc, esem, gsem, ssem) = refs
    else:
        (rows_hbm, cols_hbm, norm_hbm, x_hbm, y_out,
         rows_v, cols_v, scat_v, norm_v, nm_v, xbuf, wb, acc,
         esem, gsem, ssem) = refs

    c = lax.axis_index("c")
    s = lax.axis_index("s")
    base = c * HALF

    if first:
        @pl.when(s == 0)
        def _():
            pltpu.sync_copy(dis_hbm, dis_sp)

    _zero_fill(wb, WB)
    arow = s * 1600

    def zbody(k, _):
        pltpu.sync_copy(wb, acc.at[pl.ds(arow + k * WB, WB)])
        return 0

    lax.fori_loop(0, 20, zbody, 0)
    plsc.subcore_barrier()

    start = s * 195 + jnp.minimum(s, 5)
    cnt = 195 + jnp.where(s < 5, 1, 0)

    def body(t, _):
        par = t % 2
        u = start + t
        ew = [pltpu.async_copy(rows_hbm.at[pl.ds(2 * u, 2)],
                               rows_v.at[par], esem.at[0]),
              pltpu.async_copy(cols_hbm.at[pl.ds(2 * u, 2)],
                               cols_v.at[par], esem.at[1])]
        if first:
            ew.append(pltpu.async_copy(ew_hbm.at[pl.ds(u * SUP, SUP)],
                                       ew_v.at[par], esem.at[2]))
        else:
            ew.append(pltpu.async_copy(norm_hbm.at[pl.ds(u * SUP, SUP)],
                                       norm_v.at[par], esem.at[2]))
        for wd in ew:
            wd.wait()

        gw = [pltpu.async_copy(x_hbm.at[rows_v.at[par, k]],
                               xbuf.at[pl.ds(k * 128, 128)], gsem.at[k])
              for k in range(2)]
        dw = []
        if first:
            for k in range(2):
                dw.append(pltpu.async_copy(
                    dis_sp.at[rows_v.at[par, k]], drbuf.at[k],
                    gsem.at[2 + k]))
                dw.append(pltpu.async_copy(
                    dis_sp.at[cols_v.at[par, k]], dcbuf.at[k],
                    gsem.at[4 + k]))

        def sbody(g, _):
            nv16 = nm_v[pl.ds(g * 16, 16)]
            for l in range(16):
                sc = nv16[l]
                e = g * 16 + l
                for j in range(4):
                    sl2 = pl.ds(j * 16, 16)
                    xbuf[e, sl2] = xbuf[e, sl2] * sc
            return 0

        sw = []
        for k in range(2):
            gw[k].wait()
            if first:
                dw[2 * k].wait()
                dw[2 * k + 1].wait()
            for i in range(8):
                sl = pl.ds(i * 16, 16)
                e0 = k * 128 + i * 16
                c16 = cols_v[par, k, sl]
                lc = c16 - base
                valid = (lc >= 0) & (lc < HALF)
                if first:
                    nv = (drbuf[k, sl] * ew_v[par, pl.ds(e0, 16)]
                          * dcbuf[k, sl])
                    norm_v[par, pl.ds(e0, 16)] = nv
                else:
                    nv = norm_v[par, pl.ds(e0, 16)]
                scat_v[k, sl] = jnp.where(valid, lc, 0)
                nm_v[pl.ds(e0, 16)] = jnp.where(valid, nv, 0.0)
            lax.fori_loop(8 * k, 8 * k + 8, sbody, 0)
            sw.append(pltpu.async_copy(
                xbuf.at[pl.ds(k * 128, 128)], acc.at[scat_v.at[k]],
                ssem.at[k], add=True))

        if first:
            @pl.when(c == 0)
            def _():
                pltpu.sync_copy(norm_v.at[par],
                                norm_out.at[pl.ds(u * SUP, SUP)])

        for wd in sw:
            wd.wait()
        return 0

    lax.fori_loop(0, cnt, body, 0)
    plsc.subcore_barrier()

    def _relu_rows(n):
        def rbody(r, _):
            for j in range(4):
                sl2 = pl.ds(j * 16, 16)
                wb[r, sl2] = jnp.maximum(wb[r, sl2], 0.0)
            return 0

        lax.fori_loop(0, n, rbody, 0)

    def wbody(k, _):
        rbase = s * 1560 + k * WB
        pltpu.sync_copy(acc.at[pl.ds(rbase, WB)], wb)
        _relu_rows(WB)
        pltpu.sync_copy(wb, y_out.at[pl.ds(base + rbase, WB)])
        return 0

    lax.fori_loop(0, 19, wbody, 0)
    rbase = s * 1560 + 1520
    pltpu.sync_copy(acc.at[pl.ds(rbase, 40)], wb.at[pl.ds(0, 40)])
    _relu_rows(40)
    pltpu.sync_copy(wb.at[pl.ds(0, 40)], y_out.at[pl.ds(base + rbase, 40)])

    @pl.when(s == 15)
    def _():
        pltpu.sync_copy(acc.at[pl.ds(24960, 40)], wb.at[pl.ds(0, 40)])
        _relu_rows(40)
        pltpu.sync_copy(wb.at[pl.ds(0, 40)],
                        y_out.at[pl.ds(base + 24960, 40)])


@functools.cache
def _layer1():
    return functools.partial(
        pl.kernel,
        out_type=(jax.ShapeDtypeStruct((N, D), _f32),
                  jax.ShapeDtypeStruct((E,), _f32)),
        mesh=_mesh(),
        compiler_params=pltpu.CompilerParams(
            needs_layout_passes=False, use_tc_tiling_on_sc=False),
        scratch_types=[
            pltpu.VMEM((2, 2, 128), _i32),
            pltpu.VMEM((2, 2, 128), _i32),
            pltpu.VMEM((2, 128), _i32),
            pltpu.VMEM((2, SUP), _f32),
            pltpu.VMEM((2, SUP), _f32),
            pltpu.VMEM((SUP,), _f32),
            pltpu.VMEM((2, 128), _f32),
            pltpu.VMEM((2, 128), _f32),
            pltpu.VMEM((SUP, D), _f32),
            pltpu.VMEM((WB, D), _f32),
            pltpu.VMEM_SHARED((NPAD,), _f32),
            pltpu.VMEM_SHARED((ACC_ROWS, D), _f32),
            pltpu.SemaphoreType.DMA((3,)),
            pltpu.SemaphoreType.DMA((6,)),
            pltpu.SemaphoreType.DMA((2,)),
        ],
    )(functools.partial(_layer_body, True))


@functools.cache
def _layer2():
    return functools.partial(
        pl.kernel,
        out_type=jax.ShapeDtypeStruct((N, D), _f32),
        mesh=_mesh(),
        compiler_params=pltpu.CompilerParams(
            needs_layout_passes=False, use_tc_tiling_on_sc=False),
        scratch_types=[
            pltpu.VMEM((2, 2, 128), _i32),
            pltpu.VMEM((2, 2, 128), _i32),
            pltpu.VMEM((2, 128), _i32),
            pltpu.VMEM((2, SUP), _f32),
            pltpu.VMEM((SUP,), _f32),
            pltpu.VMEM((SUP, D), _f32),
            pltpu.VMEM((WB, D), _f32),
            pltpu.VMEM_SHARED((ACC_ROWS, D), _f32),
            pltpu.SemaphoreType.DMA((3,)),
            pltpu.SemaphoreType.DMA((6,)),
            pltpu.SemaphoreType.DMA((2,)),
        ],
    )(functools.partial(_layer_body, False))


def _final_body(x0_ref, y1_ref, y2_ref, w_ref, b_ref, out_ref):
    xs = (x0_ref[...] + y1_ref[...] + y2_ref[...]) * (1.0 / 3.0)
    out_ref[...] = lax.dot_general(
        xs, w_ref[...], (((1,), (1,)), ((), ())),
        preferred_element_type=_f32) + b_ref[...]


def _final(x0, y1, y2, W, b):
    grid = 125
    blk = N // grid
    return pl.pallas_call(
        _final_body,
        grid=(grid,),
        in_specs=[
            pl.BlockSpec((blk, D), lambda i: (i, 0)),
            pl.BlockSpec((blk, D), lambda i: (i, 0)),
            pl.BlockSpec((blk, D), lambda i: (i, 0)),
            pl.BlockSpec((D, D), lambda i: (0, 0)),
            pl.BlockSpec((1, D), lambda i: (0, 0)),
        ],
        out_specs=pl.BlockSpec((blk, D), lambda i: (i, 0)),
        out_shape=jax.ShapeDtypeStruct((N, D), _f32),
    )(x0, y1, y2, W, b.reshape(1, D))


def kernel(edge_index, edge_weight, emb_users, emb_items, W, b):
    rows1d = edge_index[0].astype(_i32)
    cols1d = edge_index[1].astype(_i32)
    rows2d = rows1d.reshape(NB, 128)
    cols2d = cols1d.reshape(NB, 128)
    x0 = jnp.concatenate([emb_users, emb_items], axis=0)

    deg_p = _deg_kernel()(cols1d, edge_weight)
    dis = _dis(deg_p)
    y1, norm1d = _layer1()(rows2d, cols2d, edge_weight, dis, x0)
    y2 = _layer2()(rows2d, cols2d, norm1d, y1)
    out = _final(x0, y1, y2, W, b)
    nu = emb_users.shape[0]
    return (out[:nu], emb_users, out[nu:], emb_items)


# speedup vs baseline: 6.6045x; 1.0003x over previous
import functools

import jax
import jax.numpy as jnp
from jax import lax
from jax.experimental import pallas as pl
from jax.experimental.pallas import tpu as pltpu
from jax.experimental.pallas import tpu_sc as plsc

N = 50000
E = 800000
D = 64
NPAD = 50048
NB = E // 128
HALF = N // 2
ACC_ROWS = 25600
WB = 80

_f32 = jnp.float32
_i32 = jnp.int32


@functools.cache
def _mesh():
    return plsc.VectorSubcoreMesh(
        core_axis_name="c", subcore_axis_name="s", num_cores=2,
        num_subcores=16)


def _bcast(v, lvec):
    return lax.gather(
        v, lvec.reshape(16, 1),
        lax.GatherDimensionNumbers(offset_dims=(),
                                   collapsed_slice_dims=(0,),
                                   start_index_map=(0,)),
        slice_sizes=(1,),
        mode=lax.GatherScatterMode.PROMISE_IN_BOUNDS)


def _zero_fill(buf, rows):
    z = jnp.zeros((16,), _f32)

    def body(r, _):
        for j in range(4):
            buf[r, pl.ds(j * 16, 16)] = z
        return 0

    lax.fori_loop(0, rows, body, 0)


@functools.cache
def _deg_kernel():
    return functools.partial(
        pl.kernel,
        out_type=jax.ShapeDtypeStruct((2 * NPAD,), _f32),
        mesh=_mesh(),
        compiler_params=pltpu.CompilerParams(
            needs_layout_passes=False, use_tc_tiling_on_sc=False),
        scratch_types=[
            pltpu.VMEM((128,), _i32),
            pltpu.VMEM((128,), _f32),
            pltpu.VMEM((3136,), _f32),
            pltpu.VMEM_SHARED((NPAD,), _f32),
        ],
    )(_deg_body)


def _deg_body(cols_hbm, ew_hbm, deg_out, col_v, ew_v, zb, deg_sp):
    c = lax.axis_index("c")
    s = lax.axis_index("s")
    z = jnp.zeros((16,), _f32)

    def zb_body(i, _):
        zb[pl.ds(i * 16, 16)] = z
        return 0

    lax.fori_loop(0, 196, zb_body, 0)
    pltpu.sync_copy(zb.at[pl.ds(0, 3128)], deg_sp.at[pl.ds(s * 3128, 3128)])
    plsc.subcore_barrier()

    w = c * 16 + s
    start = w * 195 + jnp.minimum(w, 10)
    cnt = 195 + jnp.where(w < 10, 1, 0)

    def body(t, _):
        eoff = (start + t) * 128
        pltpu.sync_copy(cols_hbm.at[pl.ds(eoff, 128)], col_v)
        pltpu.sync_copy(ew_hbm.at[pl.ds(eoff, 128)], ew_v)
        pltpu.sync_copy(ew_v, deg_sp.at[col_v], add=True)
        return 0

    lax.fori_loop(0, cnt, body, 0)
    plsc.subcore_barrier()
    pltpu.sync_copy(deg_sp.at[pl.ds(s * 3128, 3128)], zb.at[pl.ds(0, 3128)])
    pltpu.sync_copy(zb.at[pl.ds(0, 3128)],
                    deg_out.at[pl.ds(c * NPAD + s * 3128, 3128)])


def _dis_body(deg_ref, dis_ref):
    d = deg_ref[0] + deg_ref[1]
    dis_ref[...] = jnp.where(d > 0, lax.rsqrt(d), 0.0)


def _dis(deg_p):
    out = pl.pallas_call(
        _dis_body,
        out_shape=jax.ShapeDtypeStruct((391, 128), _f32),
    )(deg_p.reshape(2, 391, 128))
    return out.reshape(NPAD)


SUP = 256
NSUP = E // SUP


def _layer_body(first, *refs):
    if first:
        (rows_hbm, cols_hbm, ew_hbm, dis_hbm, x_hbm, y_out, norm_out,
         rows_v, cols_v, scat_v, ew_v, norm_v, nm_v, drbuf, dcbuf,
         xbuf, wb, dis_sp, acc, esem, gsem, ssem) = refs
    else:
        (rows_hbm, cols_hbm, norm_hbm, x_hbm, y_out,
         rows_v, cols_v, scat_v, norm_v, nm_v, xbuf, wb, acc,
         esem, gsem, ssem) = refs

    c = lax.axis_index("c")
    s = lax.axis_index("s")
    base = c * HALF

    if first:
        @pl.when(s == 0)
        def _():
            pltpu.sync_copy(dis_hbm, dis_sp)

    _zero_fill(wb, WB)
    arow = s * 1600

    def zbody(k, _):
        pltpu.sync_copy(wb, acc.at[pl.ds(arow + k * WB, WB)])
        return 0

    lax.fori_loop(0, 20, zbody, 0)
    plsc.subcore_barrier()

    start = s * 195 + jnp.minimum(s, 5)
    cnt = 195 + jnp.where(s < 5, 1, 0)

    def body(t, _):
        par = t % 2
        u = start + t
        ew = [pltpu.async_copy(rows_hbm.at[pl.ds(2 * u, 2)],
                               rows_v.at[par], esem.at[0]),
              pltpu.async_copy(cols_hbm.at[pl.ds(2 * u, 2)],
                               cols_v.at[par], esem.at[1])]
        if first:
            ew.append(pltpu.async_copy(ew_hbm.at[pl.ds(u * SUP, SUP)],
                                       ew_v.at[par], esem.at[2]))
        else:
            ew.append(pltpu.async_copy(norm_hbm.at[pl.ds(u * SUP, SUP)],
                                       norm_v.at[par], esem.at[2]))
        for wd in ew:
            wd.wait()

        gw = [pltpu.async_copy(x_hbm.at[rows_v.at[par, k]],
                               xbuf.at[pl.ds(k * 128, 128)], gsem.at[k])
              for k in range(2)]
        dw = []
        if first:
            for k in range(2):
                dw.append(pltpu.async_copy(
                    dis_sp.at[rows_v.at[par, k]], drbuf.at[k],
                    gsem.at[2 + k]))
                dw.append(pltpu.async_copy(
                    dis_sp.at[cols_v.at[par, k]], dcbuf.at[k],
                    gsem.at[4 + k]))

        zi16 = lax.iota(jnp.int32, 16) * 0

        def sbody(g, _):
            nv16 = nm_v[pl.ds(g * 16, 16)]
            for l in range(16):
                bc = _bcast(nv16, zi16 + l)
                e = g * 16 + l
                for j in range(4):
                    sl2 = pl.ds(j * 16, 16)
                    xbuf[e, sl2] = xbuf[e, sl2] * bc
            return 0

        sw = []
        for k in range(2):
            gw[k].wait()
            if first:
                dw[2 * k].wait()
                dw[2 * k + 1].wait()
            for i in range(8):
                sl = pl.ds(i * 16, 16)
                e0 = k * 128 + i * 16
                c16 = cols_v[par, k, sl]
                lc = c16 - base
                valid = (lc >= 0) & (lc < HALF)
                if first:
                    nv = (drbuf[k, sl] * ew_v[par, pl.ds(e0, 16)]
                          * dcbuf[k, sl])
                    norm_v[par, pl.ds(e0, 16)] = nv
                else:
                    nv = norm_v[par, pl.ds(e0, 16)]
                scat_v[k, sl] = jnp.where(valid, lc, 0)
                nm_v[pl.ds(e0, 16)] = jnp.where(valid, nv, 0.0)
            lax.fori_loop(8 * k, 8 * k + 8, sbody, 0)
            sw.append(pltpu.async_copy(
                xbuf.at[pl.ds(k * 128, 128)], acc.at[scat_v.at[k]],
                ssem.at[k], add=True))

        if first:
            @pl.when(c == 0)
            def _():
                pltpu.sync_copy(norm_v.at[par],
                                norm_out.at[pl.ds(u * SUP, SUP)])

        for wd in sw:
            wd.wait()
        return 0

    lax.fori_loop(0, cnt, body, 0)
    plsc.subcore_barrier()

    def _relu_rows(n):
        def rbody(r, _):
            for j in range(4):
                sl2 = pl.ds(j * 16, 16)
                wb[r, sl2] = jnp.maximum(wb[r, sl2], 0.0)
            return 0

        lax.fori_loop(0, n, rbody, 0)

    def wbody(k, _):
        rbase = s * 1560 + k * WB
        pltpu.sync_copy(acc.at[pl.ds(rbase, WB)], wb)
        _relu_rows(WB)
        pltpu.sync_copy(wb, y_out.at[pl.ds(base + rbase, WB)])
        return 0

    lax.fori_loop(0, 19, wbody, 0)
    rbase = s * 1560 + 1520
    pltpu.sync_copy(acc.at[pl.ds(rbase, 40)], wb.at[pl.ds(0, 40)])
    _relu_rows(40)
    pltpu.sync_copy(wb.at[pl.ds(0, 40)], y_out.at[pl.ds(base + rbase, 40)])

    @pl.when(s == 15)
    def _():
        pltpu.sync_copy(acc.at[pl.ds(24960, 40)], wb.at[pl.ds(0, 40)])
        _relu_rows(40)
        pltpu.sync_copy(wb.at[pl.ds(0, 40)],
                        y_out.at[pl.ds(base + 24960, 40)])


@functools.cache
def _layer1():
    return functools.partial(
        pl.kernel,
        out_type=(jax.ShapeDtypeStruct((N, D), _f32),
                  jax.ShapeDtypeStruct((E,), _f32)),
        mesh=_mesh(),
        compiler_params=pltpu.CompilerParams(
            needs_layout_passes=False, use_tc_tiling_on_sc=False),
        scratch_types=[
            pltpu.VMEM((2, 2, 128), _i32),
            pltpu.VMEM((2, 2, 128), _i32),
            pltpu.VMEM((2, 128), _i32),
            pltpu.VMEM((2, SUP), _f32),
            pltpu.VMEM((2, SUP), _f32),
            pltpu.VMEM((SUP,), _f32),
            pltpu.VMEM((2, 128), _f32),
            pltpu.VMEM((2, 128), _f32),
            pltpu.VMEM((SUP, D), _f32),
            pltpu.VMEM((WB, D), _f32),
            pltpu.VMEM_SHARED((NPAD,), _f32),
            pltpu.VMEM_SHARED((ACC_ROWS, D), _f32),
            pltpu.SemaphoreType.DMA((3,)),
            pltpu.SemaphoreType.DMA((6,)),
            pltpu.SemaphoreType.DMA((2,)),
        ],
    )(functools.partial(_layer_body, True))


@functools.cache
def _layer2():
    return functools.partial(
        pl.kernel,
        out_type=jax.ShapeDtypeStruct((N, D), _f32),
        mesh=_mesh(),
        compiler_params=pltpu.CompilerParams(
            needs_layout_passes=False, use_tc_tiling_on_sc=False),
        scratch_types=[
            pltpu.VMEM((2, 2, 128), _i32),
            pltpu.VMEM((2, 2, 128), _i32),
            pltpu.VMEM((2, 128), _i32),
            pltpu.VMEM((2, SUP), _f32),
            pltpu.VMEM((SUP,), _f32),
            pltpu.VMEM((SUP, D), _f32),
            pltpu.VMEM((WB, D), _f32),
            pltpu.VMEM_SHARED((ACC_ROWS, D), _f32),
            pltpu.SemaphoreType.DMA((3,)),
            pltpu.SemaphoreType.DMA((6,)),
            pltpu.SemaphoreType.DMA((2,)),
        ],
    )(functools.partial(_layer_body, False))


def _final_body(x0_ref, y1_ref, y2_ref, w_ref, b_ref, out_ref):
    xs = (x0_ref[...] + y1_ref[...] + y2_ref[...]) * (1.0 / 3.0)
    out_ref[...] = lax.dot_general(
        xs, w_ref[...], (((1,), (1,)), ((), ())),
        preferred_element_type=_f32) + b_ref[...]


def _final(x0, y1, y2, W, b):
    grid = 125
    blk = N // grid
    return pl.pallas_call(
        _final_body,
        grid=(grid,),
        in_specs=[
            pl.BlockSpec((blk, D), lambda i: (i, 0)),
            pl.BlockSpec((blk, D), lambda i: (i, 0)),
            pl.BlockSpec((blk, D), lambda i: (i, 0)),
            pl.BlockSpec((D, D), lambda i: (0, 0)),
            pl.BlockSpec((1, D), lambda i: (0, 0)),
        ],
        out_specs=pl.BlockSpec((blk, D), lambda i: (i, 0)),
        out_shape=jax.ShapeDtypeStruct((N, D), _f32),
    )(x0, y1, y2, W, b.reshape(1, D))


def kernel(edge_index, edge_weight, emb_users, emb_items, W, b):
    rows1d = edge_index[0].astype(_i32)
    cols1d = edge_index[1].astype(_i32)
    rows2d = rows1d.reshape(NB, 128)
    cols2d = cols1d.reshape(NB, 128)
    x0 = jnp.concatenate([emb_users, emb_items], axis=0)

    deg_p = _deg_kernel()(cols1d, edge_weight)
    dis = _dis(deg_p)
    y1, norm1d = _layer1()(rows2d, cols2d, edge_weight, dis, x0)
    y2 = _layer2()(rows2d, cols2d, norm1d, y1)
    out = _final(x0, y1, y2, W, b)
    nu = emb_users.shape[0]
    return (out[:nu], emb_users, out[nu:], emb_items)


# speedup vs baseline: 11.4891x; 1.7396x over previous
import functools

import jax
import jax.numpy as jnp
from jax import lax
from jax.experimental import pallas as pl
from jax.experimental.pallas import tpu as pltpu
from jax.experimental.pallas import tpu_sc as plsc

N = 50000
E = 800000
D = 64
NPAD = 50048
NB = E // 128
HALF = N // 2
ACC_ROWS = 25600
WB = 80

_f32 = jnp.float32
_i32 = jnp.int32


@functools.cache
def _mesh():
    return plsc.VectorSubcoreMesh(
        core_axis_name="c", subcore_axis_name="s", num_cores=2,
        num_subcores=16)


def _bcast(v, lvec):
    return lax.gather(
        v, lvec.reshape(16, 1),
        lax.GatherDimensionNumbers(offset_dims=(),
                                   collapsed_slice_dims=(0,),
                                   start_index_map=(0,)),
        slice_sizes=(1,),
        mode=lax.GatherScatterMode.PROMISE_IN_BOUNDS)


def _zero_fill(buf, rows):
    z = jnp.zeros((16,), _f32)

    def body(r, _):
        for j in range(4):
            buf[r, pl.ds(j * 16, 16)] = z
        return 0

    lax.fori_loop(0, rows, body, 0)


@functools.cache
def _deg_kernel():
    return functools.partial(
        pl.kernel,
        out_type=jax.ShapeDtypeStruct((2 * NPAD,), _f32),
        mesh=_mesh(),
        compiler_params=pltpu.CompilerParams(
            needs_layout_passes=False, use_tc_tiling_on_sc=False),
        scratch_types=[
            pltpu.VMEM((128,), _i32),
            pltpu.VMEM((128,), _f32),
            pltpu.VMEM((3136,), _f32),
            pltpu.VMEM_SHARED((NPAD,), _f32),
        ],
    )(_deg_body)


def _deg_body(cols_hbm, ew_hbm, deg_out, col_v, ew_v, zb, deg_sp):
    c = lax.axis_index("c")
    s = lax.axis_index("s")
    z = jnp.zeros((16,), _f32)

    def zb_body(i, _):
        zb[pl.ds(i * 16, 16)] = z
        return 0

    lax.fori_loop(0, 196, zb_body, 0)
    pltpu.sync_copy(zb.at[pl.ds(0, 3128)], deg_sp.at[pl.ds(s * 3128, 3128)])
    plsc.subcore_barrier()

    w = c * 16 + s
    start = w * 195 + jnp.minimum(w, 10)
    cnt = 195 + jnp.where(w < 10, 1, 0)

    def body(t, _):
        eoff = (start + t) * 128
        pltpu.sync_copy(cols_hbm.at[pl.ds(eoff, 128)], col_v)
        pltpu.sync_copy(ew_hbm.at[pl.ds(eoff, 128)], ew_v)
        pltpu.sync_copy(ew_v, deg_sp.at[col_v], add=True)
        return 0

    lax.fori_loop(0, cnt, body, 0)
    plsc.subcore_barrier()
    pltpu.sync_copy(deg_sp.at[pl.ds(s * 3128, 3128)], zb.at[pl.ds(0, 3128)])
    pltpu.sync_copy(zb.at[pl.ds(0, 3128)],
                    deg_out.at[pl.ds(c * NPAD + s * 3128, 3128)])


def _dis_body(deg_ref, dis_ref):
    d = deg_ref[0] + deg_ref[1]
    dis_ref[...] = jnp.where(d > 0, lax.rsqrt(d), 0.0)


def _dis(deg_p):
    out = pl.pallas_call(
        _dis_body,
        out_shape=jax.ShapeDtypeStruct((391, 128), _f32),
    )(deg_p.reshape(2, 391, 128))
    return out.reshape(NPAD)


SUP = 256
NSUP = E // SUP


def _layer_body(first, *refs):
    if first:
        (rows_hbm, cols_hbm, ew_hbm, dis_hbm, x_hbm, y_out, norm_out,
         rows_v, cols_v, scat_v, ew_v, norm_v, nm_v, drbuf, dcbuf,
         xbuf, wb, dis_sp, acc, esem, gsem, ssem) = refs
    else:
        (rows_hbm, cols_hbm, norm_hbm, x_hbm, y_out,
         rows_v, cols_v, scat_v, norm_v, nm_v, xbuf, wb, acc,
         esem, gsem, ssem) = refs

    c = lax.axis_index("c")
    s = lax.axis_index("s")
    base = c * HALF

    if first:
        @pl.when(s == 0)
        def _():
            pltpu.sync_copy(dis_hbm, dis_sp)

    _zero_fill(wb, WB)
    arow = s * 1600

    def zbody(k, _):
        pltpu.sync_copy(wb, acc.at[pl.ds(arow + k * WB, WB)])
        return 0

    lax.fori_loop(0, 20, zbody, 0)
    plsc.subcore_barrier()

    start = s * 195 + jnp.minimum(s, 5)
    cnt = 195 + jnp.where(s < 5, 1, 0)

    def body(t, _):
        par = t % 2
        u = start + t
        ew = [pltpu.async_copy(rows_hbm.at[pl.ds(2 * u, 2)],
                               rows_v.at[par], esem.at[0]),
              pltpu.async_copy(cols_hbm.at[pl.ds(2 * u, 2)],
                               cols_v.at[par], esem.at[1])]
        if first:
            ew.append(pltpu.async_copy(ew_hbm.at[pl.ds(u * SUP, SUP)],
                                       ew_v.at[par], esem.at[2]))
        else:
            ew.append(pltpu.async_copy(norm_hbm.at[pl.ds(u * SUP, SUP)],
                                       norm_v.at[par], esem.at[2]))
        for wd in ew:
            wd.wait()

        gw = [pltpu.async_copy(x_hbm.at[rows_v.at[par, k]],
                               xbuf.at[pl.ds(k * 128, 128)], gsem.at[k])
              for k in range(2)]
        dw = []
        if first:
            for k in range(2):
                dw.append(pltpu.async_copy(
                    dis_sp.at[rows_v.at[par, k]], drbuf.at[k],
                    gsem.at[2 + k]))
                dw.append(pltpu.async_copy(
                    dis_sp.at[cols_v.at[par, k]], dcbuf.at[k],
                    gsem.at[4 + k]))

        def scale_half(k):
            for g in range(8 * k, 8 * k + 8):
                nv16 = nm_v[pl.ds(g * 16, 16)]
                for l in range(16):
                    sc = nv16[l]
                    e = g * 16 + l
                    for j in range(4):
                        sl2 = pl.ds(j * 16, 16)
                        xbuf[e, sl2] = xbuf[e, sl2] * sc

        sw = []
        for k in range(2):
            gw[k].wait()
            if first:
                dw[2 * k].wait()
                dw[2 * k + 1].wait()
            for i in range(8):
                sl = pl.ds(i * 16, 16)
                e0 = k * 128 + i * 16
                c16 = cols_v[par, k, sl]
                lc = c16 - base
                valid = (lc >= 0) & (lc < HALF)
                if first:
                    nv = (drbuf[k, sl] * ew_v[par, pl.ds(e0, 16)]
                          * dcbuf[k, sl])
                    norm_v[par, pl.ds(e0, 16)] = nv
                else:
                    nv = norm_v[par, pl.ds(e0, 16)]
                scat_v[k, sl] = jnp.where(valid, lc, 0)
                nm_v[pl.ds(e0, 16)] = jnp.where(valid, nv, 0.0)
            scale_half(k)
            sw.append(pltpu.async_copy(
                xbuf.at[pl.ds(k * 128, 128)], acc.at[scat_v.at[k]],
                ssem.at[k], add=True))

        if first:
            @pl.when(c == 0)
            def _():
                pltpu.sync_copy(norm_v.at[par],
                                norm_out.at[pl.ds(u * SUP, SUP)])

        for wd in sw:
            wd.wait()
        return 0

    lax.fori_loop(0, cnt, body, 0)
    plsc.subcore_barrier()

    def _relu_rows(n):
        def rbody(r, _):
            for j in range(4):
                sl2 = pl.ds(j * 16, 16)
                wb[r, sl2] = jnp.maximum(wb[r, sl2], 0.0)
            return 0

        lax.fori_loop(0, n, rbody, 0)

    def wbody(k, _):
        rbase = s * 1560 + k * WB
        pltpu.sync_copy(acc.at[pl.ds(rbase, WB)], wb)
        _relu_rows(WB)
        pltpu.sync_copy(wb, y_out.at[pl.ds(base + rbase, WB)])
        return 0

    lax.fori_loop(0, 19, wbody, 0)
    rbase = s * 1560 + 1520
    pltpu.sync_copy(acc.at[pl.ds(rbase, 40)], wb.at[pl.ds(0, 40)])
    _relu_rows(40)
    pltpu.sync_copy(wb.at[pl.ds(0, 40)], y_out.at[pl.ds(base + rbase, 40)])

    @pl.when(s == 15)
    def _():
        pltpu.sync_copy(acc.at[pl.ds(24960, 40)], wb.at[pl.ds(0, 40)])
        _relu_rows(40)
        pltpu.sync_copy(wb.at[pl.ds(0, 40)],
                        y_out.at[pl.ds(base + 24960, 40)])


@functools.cache
def _layer1():
    return functools.partial(
        pl.kernel,
        out_type=(jax.ShapeDtypeStruct((N, D), _f32),
                  jax.ShapeDtypeStruct((E,), _f32)),
        mesh=_mesh(),
        compiler_params=pltpu.CompilerParams(
            needs_layout_passes=False, use_tc_tiling_on_sc=False),
        scratch_types=[
            pltpu.VMEM((2, 2, 128), _i32),
            pltpu.VMEM((2, 2, 128), _i32),
            pltpu.VMEM((2, 128), _i32),
            pltpu.VMEM((2, SUP), _f32),
            pltpu.VMEM((2, SUP), _f32),
            pltpu.VMEM((SUP,), _f32),
            pltpu.VMEM((2, 128), _f32),
            pltpu.VMEM((2, 128), _f32),
            pltpu.VMEM((SUP, D), _f32),
            pltpu.VMEM((WB, D), _f32),
            pltpu.VMEM_SHARED((NPAD,), _f32),
            pltpu.VMEM_SHARED((ACC_ROWS, D), _f32),
            pltpu.SemaphoreType.DMA((3,)),
            pltpu.SemaphoreType.DMA((6,)),
            pltpu.SemaphoreType.DMA((2,)),
        ],
    )(functools.partial(_layer_body, True))


@functools.cache
def _layer2():
    return functools.partial(
        pl.kernel,
        out_type=jax.ShapeDtypeStruct((N, D), _f32),
        mesh=_mesh(),
        compiler_params=pltpu.CompilerParams(
            needs_layout_passes=False, use_tc_tiling_on_sc=False),
        scratch_types=[
            pltpu.VMEM((2, 2, 128), _i32),
            pltpu.VMEM((2, 2, 128), _i32),
            pltpu.VMEM((2, 128), _i32),
            pltpu.VMEM((2, SUP), _f32),
            pltpu.VMEM((SUP,), _f32),
            pltpu.VMEM((SUP, D), _f32),
            pltpu.VMEM((WB, D), _f32),
            pltpu.VMEM_SHARED((ACC_ROWS, D), _f32),
            pltpu.SemaphoreType.DMA((3,)),
            pltpu.SemaphoreType.DMA((6,)),
            pltpu.SemaphoreType.DMA((2,)),
        ],
    )(functools.partial(_layer_body, False))


def _final_body(x0_ref, y1_ref, y2_ref, w_ref, b_ref, out_ref):
    xs = (x0_ref[...] + y1_ref[...] + y2_ref[...]) * (1.0 / 3.0)
    out_ref[...] = lax.dot_general(
        xs, w_ref[...], (((1,), (1,)), ((), ())),
        preferred_element_type=_f32) + b_ref[...]


def _final(x0, y1, y2, W, b):
    grid = 125
    blk = N // grid
    return pl.pallas_call(
        _final_body,
        grid=(grid,),
        in_specs=[
            pl.BlockSpec((blk, D), lambda i: (i, 0)),
            pl.BlockSpec((blk, D), lambda i: (i, 0)),
            pl.BlockSpec((blk, D), lambda i: (i, 0)),
            pl.BlockSpec((D, D), lambda i: (0, 0)),
            pl.BlockSpec((1, D), lambda i: (0, 0)),
        ],
        out_specs=pl.BlockSpec((blk, D), lambda i: (i, 0)),
        out_shape=jax.ShapeDtypeStruct((N, D), _f32),
    )(x0, y1, y2, W, b.reshape(1, D))


def kernel(edge_index, edge_weight, emb_users, emb_items, W, b):
    rows1d = edge_index[0].astype(_i32)
    cols1d = edge_index[1].astype(_i32)
    rows2d = rows1d.reshape(NB, 128)
    cols2d = cols1d.reshape(NB, 128)
    x0 = jnp.concatenate([emb_users, emb_items], axis=0)

    deg_p = _deg_kernel()(cols1d, edge_weight)
    dis = _dis(deg_p)
    y1, norm1d = _layer1()(rows2d, cols2d, edge_weight, dis, x0)
    y2 = _layer2()(rows2d, cols2d, norm1d, y1)
    out = _final(x0, y1, y2, W, b)
    nu = emb_users.shape[0]
    return (out[:nu], emb_users, out[nu:], emb_items)


# speedup vs baseline: 12.0920x; 1.0525x over previous
import functools

import jax
import jax.numpy as jnp
from jax import lax
from jax.experimental import pallas as pl
from jax.experimental.pallas import tpu as pltpu
from jax.experimental.pallas import tpu_sc as plsc

N = 50000
E = 800000
D = 64
NPAD = 50048
NB = E // 128
HALF = N // 2
ACC_ROWS = 25600
WB = 80

_f32 = jnp.float32
_i32 = jnp.int32


@functools.cache
def _mesh():
    return plsc.VectorSubcoreMesh(
        core_axis_name="c", subcore_axis_name="s", num_cores=2,
        num_subcores=16)


def _bcast(v, lvec):
    return lax.gather(
        v, lvec.reshape(16, 1),
        lax.GatherDimensionNumbers(offset_dims=(),
                                   collapsed_slice_dims=(0,),
                                   start_index_map=(0,)),
        slice_sizes=(1,),
        mode=lax.GatherScatterMode.PROMISE_IN_BOUNDS)


def _zero_fill(buf, rows):
    z = jnp.zeros((16,), _f32)

    def body(r, _):
        for j in range(4):
            buf[r, pl.ds(j * 16, 16)] = z
        return 0

    lax.fori_loop(0, rows, body, 0)


@functools.cache
def _deg_kernel():
    return functools.partial(
        pl.kernel,
        out_type=jax.ShapeDtypeStruct((2 * NPAD,), _f32),
        mesh=_mesh(),
        compiler_params=pltpu.CompilerParams(
            needs_layout_passes=False, use_tc_tiling_on_sc=False),
        scratch_types=[
            pltpu.VMEM((128,), _i32),
            pltpu.VMEM((128,), _f32),
            pltpu.VMEM((3136,), _f32),
            pltpu.VMEM_SHARED((NPAD,), _f32),
            pltpu.SemaphoreType.DMA((2,)),
        ],
    )(_deg_body)


def _deg_body(cols_hbm, ew_hbm, deg_out, col_v, ew_v, zb, deg_sp, dsem):
    c = lax.axis_index("c")
    s = lax.axis_index("s")
    z = jnp.zeros((16,), _f32)

    def zb_body(i, _):
        zb[pl.ds(i * 16, 16)] = z
        return 0

    lax.fori_loop(0, 196, zb_body, 0)
    pltpu.sync_copy(zb.at[pl.ds(0, 3128)], deg_sp.at[pl.ds(s * 3128, 3128)])
    plsc.subcore_barrier()

    w = c * 16 + s
    start = w * 195 + jnp.minimum(w, 10)
    cnt = 195 + jnp.where(w < 10, 1, 0)

    def body(t, _):
        eoff = (start + t) * 128
        w1 = pltpu.async_copy(cols_hbm.at[pl.ds(eoff, 128)], col_v,
                              dsem.at[0])
        w2 = pltpu.async_copy(ew_hbm.at[pl.ds(eoff, 128)], ew_v, dsem.at[1])
        w1.wait()
        w2.wait()
        pltpu.sync_copy(ew_v, deg_sp.at[col_v], add=True)
        return 0

    lax.fori_loop(0, cnt, body, 0)
    plsc.subcore_barrier()
    pltpu.sync_copy(deg_sp.at[pl.ds(s * 3128, 3128)], zb.at[pl.ds(0, 3128)])
    pltpu.sync_copy(zb.at[pl.ds(0, 3128)],
                    deg_out.at[pl.ds(c * NPAD + s * 3128, 3128)])


def _dis_body(deg_ref, dis_ref):
    d = deg_ref[0] + deg_ref[1]
    dis_ref[...] = jnp.where(d > 0, lax.rsqrt(d), 0.0)


def _dis(deg_p):
    out = pl.pallas_call(
        _dis_body,
        out_shape=jax.ShapeDtypeStruct((391, 128), _f32),
    )(deg_p.reshape(2, 391, 128))
    return out.reshape(NPAD)


SUP = 256
NSUP = E // SUP


def _layer_body(first, *refs):
    if first:
        (rows_hbm, cols_hbm, ew_hbm, dis_hbm, x_hbm, y_out, norm_out,
         rows_v, cols_v, scat_v, ew_v, norm_v, nm_v, drbuf, dcbuf,
         xbuf, wb, dis_sp, acc, esem, gsem, ssem) = refs
    else:
        (rows_hbm, cols_hbm, norm_hbm, x_hbm, y_out,
         rows_v, cols_v, scat_v, norm_v, nm_v, xbuf, wb, acc,
         esem, gsem, ssem) = refs

    c = lax.axis_index("c")
    s = lax.axis_index("s")
    base = c * HALF

    if first:
        @pl.when(s == 0)
        def _():
            pltpu.sync_copy(dis_hbm, dis_sp)

    _zero_fill(wb, WB)
    arow = s * 1600

    def zbody(k, _):
        pltpu.sync_copy(wb, acc.at[pl.ds(arow + k * WB, WB)])
        return 0

    lax.fori_loop(0, 20, zbody, 0)
    plsc.subcore_barrier()

    start = s * 195 + jnp.minimum(s, 5)
    cnt = 195 + jnp.where(s < 5, 1, 0)

    def body(t, _):
        par = t % 2
        u = start + t
        ew = [pltpu.async_copy(rows_hbm.at[pl.ds(2 * u, 2)],
                               rows_v.at[par], esem.at[0]),
              pltpu.async_copy(cols_hbm.at[pl.ds(2 * u, 2)],
                               cols_v.at[par], esem.at[1])]
        if first:
            ew.append(pltpu.async_copy(ew_hbm.at[pl.ds(u * SUP, SUP)],
                                       ew_v.at[par], esem.at[2]))
        else:
            ew.append(pltpu.async_copy(norm_hbm.at[pl.ds(u * SUP, SUP)],
                                       norm_v.at[par], esem.at[2]))
        for wd in ew:
            wd.wait()

        gw = [pltpu.async_copy(x_hbm.at[rows_v.at[par, k]],
                               xbuf.at[pl.ds(k * 128, 128)], gsem.at[k])
              for k in range(2)]
        dw = []
        if first:
            for k in range(2):
                dw.append(pltpu.async_copy(
                    dis_sp.at[rows_v.at[par, k]], drbuf.at[k],
                    gsem.at[2 + k]))
                dw.append(pltpu.async_copy(
                    dis_sp.at[cols_v.at[par, k]], dcbuf.at[k],
                    gsem.at[4 + k]))

        def scale_half(k):
            for g in range(8 * k, 8 * k + 8):
                nv16 = nm_v[pl.ds(g * 16, 16)]
                for l in range(16):
                    sc = nv16[l]
                    e = g * 16 + l
                    for j in range(4):
                        sl2 = pl.ds(j * 16, 16)
                        xbuf[e, sl2] = xbuf[e, sl2] * sc

        sw = []
        for k in range(2):
            gw[k].wait()
            if first:
                dw[2 * k].wait()
                dw[2 * k + 1].wait()
            for i in range(8):
                sl = pl.ds(i * 16, 16)
                e0 = k * 128 + i * 16
                c16 = cols_v[par, k, sl]
                lc = c16 - base
                valid = (lc >= 0) & (lc < HALF)
                if first:
                    nv = (drbuf[k, sl] * ew_v[par, pl.ds(e0, 16)]
                          * dcbuf[k, sl])
                    norm_v[par, pl.ds(e0, 16)] = nv
                else:
                    nv = norm_v[par, pl.ds(e0, 16)]
                scat_v[k, sl] = jnp.where(valid, lc, 0)
                nm_v[pl.ds(e0, 16)] = jnp.where(valid, nv, 0.0)
            scale_half(k)
            sw.append(pltpu.async_copy(
                xbuf.at[pl.ds(k * 128, 128)], acc.at[scat_v.at[k]],
                ssem.at[k], add=True))

        if first:
            @pl.when(c == 0)
            def _():
                pltpu.sync_copy(norm_v.at[par],
                                norm_out.at[pl.ds(u * SUP, SUP)])

        for wd in sw:
            wd.wait()
        return 0

    lax.fori_loop(0, cnt, body, 0)
    plsc.subcore_barrier()

    def _relu_rows(n):
        def rbody(r, _):
            for j in range(4):
                sl2 = pl.ds(j * 16, 16)
                wb[r, sl2] = jnp.maximum(wb[r, sl2], 0.0)
            return 0

        lax.fori_loop(0, n, rbody, 0)

    def wbody(k, _):
        rbase = s * 1560 + k * WB
        pltpu.sync_copy(acc.at[pl.ds(rbase, WB)], wb)
        _relu_rows(WB)
        pltpu.sync_copy(wb, y_out.at[pl.ds(base + rbase, WB)])
        return 0

    lax.fori_loop(0, 19, wbody, 0)
    rbase = s * 1560 + 1520
    pltpu.sync_copy(acc.at[pl.ds(rbase, 40)], wb.at[pl.ds(0, 40)])
    _relu_rows(40)
    pltpu.sync_copy(wb.at[pl.ds(0, 40)], y_out.at[pl.ds(base + rbase, 40)])

    @pl.when(s == 15)
    def _():
        pltpu.sync_copy(acc.at[pl.ds(24960, 40)], wb.at[pl.ds(0, 40)])
        _relu_rows(40)
        pltpu.sync_copy(wb.at[pl.ds(0, 40)],
                        y_out.at[pl.ds(base + 24960, 40)])


@functools.cache
def _layer1():
    return functools.partial(
        pl.kernel,
        out_type=(jax.ShapeDtypeStruct((N, D), _f32),
                  jax.ShapeDtypeStruct((E,), _f32)),
        mesh=_mesh(),
        compiler_params=pltpu.CompilerParams(
            needs_layout_passes=False, use_tc_tiling_on_sc=False),
        scratch_types=[
            pltpu.VMEM((2, 2, 128), _i32),
            pltpu.VMEM((2, 2, 128), _i32),
            pltpu.VMEM((2, 128), _i32),
            pltpu.VMEM((2, SUP), _f32),
            pltpu.VMEM((2, SUP), _f32),
            pltpu.VMEM((SUP,), _f32),
            pltpu.VMEM((2, 128), _f32),
            pltpu.VMEM((2, 128), _f32),
            pltpu.VMEM((SUP, D), _f32),
            pltpu.VMEM((WB, D), _f32),
            pltpu.VMEM_SHARED((NPAD,), _f32),
            pltpu.VMEM_SHARED((ACC_ROWS, D), _f32),
            pltpu.SemaphoreType.DMA((3,)),
            pltpu.SemaphoreType.DMA((6,)),
            pltpu.SemaphoreType.DMA((2,)),
        ],
    )(functools.partial(_layer_body, True))


@functools.cache
def _layer2():
    return functools.partial(
        pl.kernel,
        out_type=jax.ShapeDtypeStruct((N, D), _f32),
        mesh=_mesh(),
        compiler_params=pltpu.CompilerParams(
            needs_layout_passes=False, use_tc_tiling_on_sc=False),
        scratch_types=[
            pltpu.VMEM((2, 2, 128), _i32),
            pltpu.VMEM((2, 2, 128), _i32),
            pltpu.VMEM((2, 128), _i32),
            pltpu.VMEM((2, SUP), _f32),
            pltpu.VMEM((SUP,), _f32),
            pltpu.VMEM((SUP, D), _f32),
            pltpu.VMEM((WB, D), _f32),
            pltpu.VMEM_SHARED((ACC_ROWS, D), _f32),
            pltpu.SemaphoreType.DMA((3,)),
            pltpu.SemaphoreType.DMA((6,)),
            pltpu.SemaphoreType.DMA((2,)),
        ],
    )(functools.partial(_layer_body, False))


def _final_body(x0_ref, y1_ref, y2_ref, w_ref, b_ref, out_ref):
    xs = (x0_ref[...] + y1_ref[...] + y2_ref[...]) * (1.0 / 3.0)
    out_ref[...] = lax.dot_general(
        xs, w_ref[...], (((1,), (1,)), ((), ())),
        preferred_element_type=_f32) + b_ref[...]


def _final(x0, y1, y2, W, b):
    grid = 125
    blk = N // grid
    return pl.pallas_call(
        _final_body,
        grid=(grid,),
        in_specs=[
            pl.BlockSpec((blk, D), lambda i: (i, 0)),
            pl.BlockSpec((blk, D), lambda i: (i, 0)),
            pl.BlockSpec((blk, D), lambda i: (i, 0)),
            pl.BlockSpec((D, D), lambda i: (0, 0)),
            pl.BlockSpec((1, D), lambda i: (0, 0)),
        ],
        out_specs=pl.BlockSpec((blk, D), lambda i: (i, 0)),
        out_shape=jax.ShapeDtypeStruct((N, D), _f32),
    )(x0, y1, y2, W, b.reshape(1, D))


def kernel(edge_index, edge_weight, emb_users, emb_items, W, b):
    rows1d = edge_index[0].astype(_i32)
    cols1d = edge_index[1].astype(_i32)
    rows2d = rows1d.reshape(NB, 128)
    cols2d = cols1d.reshape(NB, 128)
    x0 = jnp.concatenate([emb_users, emb_items], axis=0)

    deg_p = _deg_kernel()(cols1d, edge_weight)
    dis = _dis(deg_p)
    y1, norm1d = _layer1()(rows2d, cols2d, edge_weight, dis, x0)
    y2 = _layer2()(rows2d, cols2d, norm1d, y1)
    out = _final(x0, y1, y2, W, b)
    nu = emb_users.shape[0]
    return (out[:nu], emb_users, out[nu:], emb_items)
